# Initial kernel scaffold; baseline (speedup 1.0000x reference)
import jax
import jax.numpy as jnp
from jax.experimental import pallas as pl


def kernel(x, edge_index, W, b):
    raise NotImplementedError("write your pallas kernel here")


# speedup vs baseline: 12.6043x; 12.6043x over previous
import functools

import jax
import jax.numpy as jnp
from jax import lax
from jax.experimental import pallas as pl
from jax.experimental.pallas import tpu as pltpu
from jax.experimental.pallas import tpu_sc as plsc

_NC = 2
_NS = 16
_NW = _NC * _NS

_N = 10000
_D = 128
_DH = _D // _NC
_B = 128
_N_PAD = 10240
_RPT = _N_PAD // _NS
_RPT_LAST = _N - (_NS - 1) * _RPT
_DW = 16

_NBD = 80
_NBP = 158

_mesh = plsc.VectorSubcoreMesh(core_axis_name="c", subcore_axis_name="s", num_cores=_NC, num_subcores=_NS)


@functools.partial(
    pl.kernel,
    out_type=jax.ShapeDtypeStruct((_NC, _N, _DW), jnp.float32),
    mesh=_mesh,
    scratch_types=[
        pltpu.VMEM((_NBD, _B), jnp.int32),
        pltpu.VMEM((_B, _DW), jnp.float32),
        pltpu.VMEM_SHARED((_N_PAD, _DW), jnp.float32),
    ],
    compiler_params=pltpu.CompilerParams(use_tc_tiling_on_sc=False),
)
def _deg_call(colsd_hbm, ones_hbm, zerow_hbm, out_hbm, cols_v, ones_v, dacc):
  cid = lax.axis_index("c")
  sid = lax.axis_index("s")
  wid = sid * _NC + cid
  pltpu.sync_copy(colsd_hbm.at[wid], cols_v)
  pltpu.sync_copy(ones_hbm, ones_v)
  pltpu.sync_copy(zerow_hbm, dacc.at[pl.ds(sid * _RPT, _RPT)])
  plsc.subcore_barrier()

  def step(j, c):
    pltpu.sync_copy(ones_v, dacc.at[cols_v.at[j]], add=True)
    return c

  lax.fori_loop(0, _NBD, step, 0)
  plsc.subcore_barrier()
  base = sid * _RPT

  @pl.when(sid < _NS - 1)
  def _():
    pltpu.sync_copy(dacc.at[pl.ds(base, _RPT)],
                    out_hbm.at[cid, pl.ds(base, _RPT)])

  @pl.when(sid == _NS - 1)
  def _():
    pltpu.sync_copy(dacc.at[pl.ds(base, _RPT_LAST)],
                    out_hbm.at[cid, pl.ds(base, _RPT_LAST)])


@functools.partial(
    pl.kernel,
    out_type=jax.ShapeDtypeStruct((_NC, _N, _DH), jnp.float32),
    mesh=_mesh,
    scratch_types=[
        pltpu.VMEM((_NBP + 2, _B), jnp.int32),
        pltpu.VMEM((_NBP, _B), jnp.int32),
        pltpu.VMEM((_B, _DH), jnp.float32),
        pltpu.VMEM((_B, _DH), jnp.float32),
        pltpu.VMEM_SHARED((_N_PAD, _DH), jnp.float32),
        pltpu.SemaphoreType.DMA,
        pltpu.SemaphoreType.DMA,
    ],
    compiler_params=pltpu.CompilerParams(use_tc_tiling_on_sc=False),
)
def _prop_call(ws_hbm, rowsp_hbm, colsp_hbm, zeroh_hbm, out_hbm,
               rows_v, cols_v, buf0, buf1, acc, sem0, sem1):
  cid = lax.axis_index("c")
  sid = lax.axis_index("s")
  w_half = ws_hbm.at[cid]
  pltpu.sync_copy(rowsp_hbm.at[sid], rows_v)
  pltpu.sync_copy(colsp_hbm.at[sid], cols_v)
  pltpu.sync_copy(zeroh_hbm, acc.at[pl.ds(sid * _RPT, _RPT)])
  plsc.subcore_barrier()

  pltpu.async_copy(w_half.at[rows_v.at[0]], buf0, sem0)
  pltpu.async_copy(w_half.at[rows_v.at[1]], buf1, sem1)

  def step(i, c):
    j = 2 * i
    pltpu.make_async_copy(w_half.at[rows_v.at[j]], buf0, sem0).wait()
    pltpu.sync_copy(buf0, acc.at[cols_v.at[j]], add=True)
    pltpu.async_copy(w_half.at[rows_v.at[j + 2]], buf0, sem0)
    pltpu.make_async_copy(w_half.at[rows_v.at[j + 1]], buf1, sem1).wait()
    pltpu.sync_copy(buf1, acc.at[cols_v.at[j + 1]], add=True)
    pltpu.async_copy(w_half.at[rows_v.at[j + 3]], buf1, sem1)
    return c

  lax.fori_loop(0, _NBP // 2, step, 0)
  pltpu.make_async_copy(w_half.at[rows_v.at[_NBP]], buf0, sem0).wait()
  pltpu.make_async_copy(w_half.at[rows_v.at[_NBP + 1]], buf1, sem1).wait()
  plsc.subcore_barrier()
  base = sid * _RPT

  @pl.when(sid < _NS - 1)
  def _():
    pltpu.sync_copy(acc.at[pl.ds(base, _RPT)],
                    out_hbm.at[cid, pl.ds(base, _RPT)])

  @pl.when(sid == _NS - 1)
  def _():
    pltpu.sync_copy(acc.at[pl.ds(base, _RPT_LAST)],
                    out_hbm.at[cid, pl.ds(base, _RPT_LAST)])


def _prep_call(dega, degb, x):
  def body(dega_ref, degb_ref, x_ref, w0_ref, s_ref, s2_ref):
    deg = dega_ref[:, 0:1] + degb_ref[:, 0:1] + 1.0
    dinv = jnp.where(deg > 0, lax.rsqrt(deg), 0.0)
    dinv2 = jnp.where(deg > 0, 1.0 / deg, 0.0)
    w0 = x_ref[...] * dinv
    w0_ref[...] = jnp.stack([w0[:, :_DH], w0[:, _DH:]])
    s_ref[...] = jnp.broadcast_to(dinv, (_N, _DH))
    s2_ref[...] = jnp.broadcast_to(dinv2, (_N, _DH))

  return pl.pallas_call(
      body,
      out_shape=(
          jax.ShapeDtypeStruct((_NC, _N, _DH), jnp.float32),
          jax.ShapeDtypeStruct((_N, _DH), jnp.float32),
          jax.ShapeDtypeStruct((_N, _DH), jnp.float32),
      ),
  )(dega, degb, x)


def _scale_call(t, w_prev, s2):
  def body(t_ref, w_ref, s2_ref, o_ref):
    o_ref[...] = s2_ref[...][None] * (t_ref[...] + w_ref[...])

  return pl.pallas_call(
      body, out_shape=jax.ShapeDtypeStruct((_NC, _N, _DH), jnp.float32)
  )(t, w_prev, s2)


def _final_call(t, w_prev, s, w_mat, bias):
  def body(t_ref, w_ref, s_ref, wm_ref, b_ref, o_ref):
    h = s_ref[...][None] * (t_ref[...] + w_ref[...])
    wm = wm_ref[...]
    o_ref[...] = (
        lax.dot_general(h[0], wm[:, :_DH], (((1,), (1,)), ((), ())),
                        preferred_element_type=jnp.float32)
        + lax.dot_general(h[1], wm[:, _DH:], (((1,), (1,)), ((), ())),
                          preferred_element_type=jnp.float32)
        + b_ref[...]
    )

  return pl.pallas_call(
      body, out_shape=jax.ShapeDtypeStruct((_N, _D), jnp.float32)
  )(t, w_prev, s, w_mat, bias)


def kernel(x, edge_index, W, b):
  row = edge_index[0].astype(jnp.int32)
  col = edge_index[1].astype(jnp.int32)
  e = row.shape[0]

  per_w = e // _NW
  padd = _NBD * _B - per_w
  colsd = jnp.pad(col.reshape(_NW, per_w), ((0, 0), (0, padd)),
                  constant_values=_N).reshape(_NW, _NBD, _B)

  per_t = e // _NS
  padp = _NBP * _B - per_t
  rowsp = jnp.pad(row.reshape(_NS, per_t), ((0, 0), (0, padp)))
  colsp = jnp.pad(col.reshape(_NS, per_t), ((0, 0), (0, padp)),
                  constant_values=_N)
  rowsp = jnp.pad(rowsp.reshape(_NS, _NBP, _B), ((0, 0), (0, 2), (0, 0)))
  colsp = colsp.reshape(_NS, _NBP, _B)

  zeros_h = jnp.zeros((_RPT, _DH), jnp.float32)
  zeros_w = jnp.zeros((_RPT, _DW), jnp.float32)
  ones_b = jnp.ones((_B, _DW), jnp.float32)
  bias = b.reshape(1, _D)

  dacc = _deg_call(colsd, ones_b, zeros_w)
  ws, s_h, s2_h = _prep_call(dacc[0], dacc[1], x)

  t = None
  for k in range(3):
    t = _prop_call(ws, rowsp, colsp, zeros_h)
    if k < 2:
      ws = _scale_call(t, ws, s2_h)
  return _final_call(t, ws, s_h, W, bias)
